# Initial kernel scaffold; baseline (speedup 1.0000x reference)
import jax
import jax.numpy as jnp
from jax.experimental import pallas as pl


def kernel(path_index, node_emb, W1, att_src1, att_dst1, W2, att_src2, att_dst2):
    raise NotImplementedError("write your pallas kernel here")


# speedup vs baseline: 25.0155x; 25.0155x over previous
import functools

import jax
import jax.numpy as jnp
from jax import lax
from jax.experimental import pallas as pl
from jax.experimental.pallas import tpu as pltpu
from jax.experimental.pallas import tpu_sc as plsc

N = 10000
NP = 10240
E = 320000
EPT = 20480
EP = 16 * EPT
NROWS = EP // 128
HW = 16
BN = 1280
MACROS = EPT // 512
RPT = NP // 16


def _dense1_body(x_ref, w_ref, as_ref, ad_ref, h0_ref, h1_ref, at_s_ref, at_d_ref):
    h = jnp.dot(x_ref[...], w_ref[...], preferred_element_type=jnp.float32)
    h0_ref[...] = h[:, :64]
    h1_ref[...] = h[:, 64:]
    at_s_ref[...] = jnp.dot(h, as_ref[...], preferred_element_type=jnp.float32)
    at_d_ref[...] = jnp.dot(h, ad_ref[...], preferred_element_type=jnp.float32)


def _dense2_body(o0_ref, o1_ref, wa_ref, wb_ref, as_ref, ad_ref,
                 h0_ref, h1_ref, h2_ref, h3_ref, at_s_ref, at_d_ref):
    v0 = o0_ref[...]
    v1 = o1_ref[...]
    xa = jnp.where(v0 > 0, v0, jnp.exp(jnp.minimum(v0, 0.0)) - 1.0)
    xb = jnp.where(v1 > 0, v1, jnp.exp(jnp.minimum(v1, 0.0)) - 1.0)
    h = (jnp.dot(xa, wa_ref[...], preferred_element_type=jnp.float32)
         + jnp.dot(xb, wb_ref[...], preferred_element_type=jnp.float32))
    h0_ref[...] = h[:, 0:128]
    h1_ref[...] = h[:, 128:256]
    h2_ref[...] = h[:, 256:384]
    h3_ref[...] = h[:, 384:512]
    at_s_ref[...] = jnp.dot(h, as_ref[...], preferred_element_type=jnp.float32)
    at_d_ref[...] = jnp.dot(h, ad_ref[...], preferred_element_type=jnp.float32)


def _dense1(x, w1, a1s, a1d):
    return pl.pallas_call(
        _dense1_body,
        grid=(NP // BN,),
        in_specs=[
            pl.BlockSpec((BN, 128), lambda i: (i, 0)),
            pl.BlockSpec((128, 128), lambda i: (0, 0)),
            pl.BlockSpec((128, HW), lambda i: (0, 0)),
            pl.BlockSpec((128, HW), lambda i: (0, 0)),
        ],
        out_specs=[
            pl.BlockSpec((BN, 64), lambda i: (i, 0)),
            pl.BlockSpec((BN, 64), lambda i: (i, 0)),
            pl.BlockSpec((BN, HW), lambda i: (i, 0)),
            pl.BlockSpec((BN, HW), lambda i: (i, 0)),
        ],
        out_shape=[
            jax.ShapeDtypeStruct((NP, 64), jnp.float32),
            jax.ShapeDtypeStruct((NP, 64), jnp.float32),
            jax.ShapeDtypeStruct((NP, HW), jnp.float32),
            jax.ShapeDtypeStruct((NP, HW), jnp.float32),
        ],
    )(x, w1, a1s, a1d)


def _dense2(o0, o1, w2a, w2b, a2s, a2d):
    return pl.pallas_call(
        _dense2_body,
        grid=(NP // BN,),
        in_specs=[
            pl.BlockSpec((BN, 64), lambda i: (i, 0)),
            pl.BlockSpec((BN, 64), lambda i: (i, 0)),
            pl.BlockSpec((64, 512), lambda i: (0, 0)),
            pl.BlockSpec((64, 512), lambda i: (0, 0)),
            pl.BlockSpec((512, HW), lambda i: (0, 0)),
            pl.BlockSpec((512, HW), lambda i: (0, 0)),
        ],
        out_specs=[
            pl.BlockSpec((BN, 128), lambda i: (i, 0)),
            pl.BlockSpec((BN, 128), lambda i: (i, 0)),
            pl.BlockSpec((BN, 128), lambda i: (i, 0)),
            pl.BlockSpec((BN, 128), lambda i: (i, 0)),
            pl.BlockSpec((BN, HW), lambda i: (i, 0)),
            pl.BlockSpec((BN, HW), lambda i: (i, 0)),
        ],
        out_shape=[
            jax.ShapeDtypeStruct((NP, 128), jnp.float32),
            jax.ShapeDtypeStruct((NP, 128), jnp.float32),
            jax.ShapeDtypeStruct((NP, 128), jnp.float32),
            jax.ShapeDtypeStruct((NP, 128), jnp.float32),
            jax.ShapeDtypeStruct((NP, HW), jnp.float32),
            jax.ShapeDtypeStruct((NP, HW), jnp.float32),
        ],
    )(o0, o1, w2a, w2b, a2s, a2d)


def _make_edge_kernel(k_cols, npass, eb_m):
    ntbl = 2 * npass
    hpp = 4 // npass
    cph = k_cols // hpp
    nblk = eb_m // 128
    nmacro = EPT // eb_m

    mesh = plsc.VectorSubcoreMesh(core_axis_name="c", subcore_axis_name="s")

    def body(src_ref, dst_ref, as_ref, ad_ref, *rest):
        htbls = rest[:ntbl]
        outs = rest[ntbl:2 * ntbl]
        ex_ref = rest[2 * ntbl]
        (spm_den, spm_out, idx_s, idx_d, ab, bb, exb, hb, sem_a, sem_b) = \
            rest[2 * ntbl + 1:]
        c = lax.axis_index("c")
        s = lax.axis_index("s")
        row0 = s * (EPT // 128)
        zero16 = jnp.zeros((16,), jnp.float32)

        def zero_exb(i, carry):
            exb[i] = zero16
            return carry

        def zero_hb(i, carry):
            for m in range(k_cols // 16):
                hb[i, pl.ds(16 * m, 16)] = zero16
            return carry

        def zero_spm(buf, spm):
            r = 0
            while r < RPT:
                ch = min(eb_m, RPT - r)
                pltpu.sync_copy(buf.at[pl.ds(0, ch)],
                                spm.at[pl.ds(nr0 + r, ch)])
                r += ch

        nr0 = s * RPT
        lax.fori_loop(0, eb_m, zero_exb, 0)
        lax.fori_loop(0, eb_m, zero_hb, 0)
        zero_spm(exb, spm_den)
        zero_spm(hb, spm_out)
        plsc.subcore_barrier()

        def macro_a(m, carry):
            r = row0 + m * nblk
            ebase = r * 128
            pltpu.sync_copy(src_ref.at[pl.ds(r, nblk)], idx_s)
            pltpu.sync_copy(dst_ref.at[pl.ds(r, nblk)], idx_d)
            cps = [pltpu.async_copy(as_ref.at[idx_s.at[k]],
                                    ab.at[pl.ds(128 * k, 128)], sem_a)
                   for k in range(nblk)]
            cpd = [pltpu.async_copy(ad_ref.at[idx_d.at[k]],
                                    bb.at[pl.ds(128 * k, 128)], sem_b)
                   for k in range(nblk)]
            for cp in cps + cpd:
                cp.wait()

            def edge(i, carry2):
                t = ab[i] + bb[i]
                exb[i] = jnp.exp(jnp.maximum(t, 0.2 * t))
                return carry2

            lax.fori_loop(0, eb_m, edge, 0)
            for k in range(nblk):
                pltpu.sync_copy(exb.at[pl.ds(128 * k, 128)],
                                spm_den.at[idx_d.at[k]], add=True)
            pltpu.sync_copy(exb, ex_ref.at[c, pl.ds(ebase, eb_m)])
            return carry

        lax.fori_loop(0, nmacro, macro_a, 0)
        plsc.subcore_barrier()

        for p in range(npass):
            def macro_b(m, carry, p=p):
                r = row0 + m * nblk
                ebase = r * 128
                pltpu.sync_copy(src_ref.at[pl.ds(r, nblk)], idx_s)
                pltpu.sync_copy(dst_ref.at[pl.ds(r, nblk)], idx_d)
                for cc in range(2):
                    @pl.when(c == cc)
                    def _():
                        cph_ = [pltpu.async_copy(
                            htbls[cc * npass + p].at[idx_s.at[k]],
                            hb.at[pl.ds(128 * k, 128)], sem_a)
                            for k in range(nblk)]
                        for cp in cph_:
                            cp.wait()
                cpd = [pltpu.async_copy(spm_den.at[idx_d.at[k]],
                                        bb.at[pl.ds(128 * k, 128)], sem_b)
                       for k in range(nblk)]
                for cp in cpd:
                    cp.wait()
                pltpu.sync_copy(ex_ref.at[c, pl.ds(ebase, eb_m)], exb)

                def edge(i, carry2):
                    a_row = exb[i] / (bb[i] + 1e-16)
                    for j in range(hpp):
                        lane = 4 * c + p * hpp + j
                        sv = jnp.take_along_axis(
                            a_row, jnp.full((16,), lane, jnp.int32), axis=0)
                        for mm in range(cph // 16):
                            off = j * cph + mm * 16
                            hb[i, pl.ds(off, 16)] = hb[i, pl.ds(off, 16)] * sv
                    return carry2

                lax.fori_loop(0, eb_m, edge, 0)
                for k in range(nblk):
                    pltpu.sync_copy(hb.at[pl.ds(128 * k, 128)],
                                    spm_out.at[idx_d.at[k]], add=True)
                return carry

            lax.fori_loop(0, nmacro, macro_b, 0)
            plsc.subcore_barrier()
            for cc in range(2):
                @pl.when(c == cc)
                def _():
                    pltpu.sync_copy(spm_out.at[pl.ds(nr0, RPT)],
                                    outs[cc * npass + p].at[pl.ds(nr0, RPT)])
            if p + 1 < npass:
                lax.fori_loop(0, eb_m, zero_hb, 0)
                zero_spm(hb, spm_out)
                plsc.subcore_barrier()

    out_type = ([jax.ShapeDtypeStruct((NP, k_cols), jnp.float32)] * ntbl
                + [jax.ShapeDtypeStruct((2, EP, HW), jnp.float32)])
    scratch = [
        pltpu.VMEM_SHARED((NP, HW), jnp.float32),
        pltpu.VMEM_SHARED((NP, k_cols), jnp.float32),
        pltpu.VMEM((nblk, 128), jnp.int32),
        pltpu.VMEM((nblk, 128), jnp.int32),
        pltpu.VMEM((eb_m, HW), jnp.float32),
        pltpu.VMEM((eb_m, HW), jnp.float32),
        pltpu.VMEM((eb_m, HW), jnp.float32),
        pltpu.VMEM((eb_m, k_cols), jnp.float32),
        pltpu.SemaphoreType.DMA,
        pltpu.SemaphoreType.DMA,
    ]
    return pl.kernel(
        body, out_type=out_type, mesh=mesh, scratch_types=scratch,
        compiler_params=pltpu.CompilerParams(use_tc_tiling_on_sc=False))


_edge_l1 = _make_edge_kernel(64, 1, 512)
_edge_l2 = _make_edge_kernel(128, 2, 128)


def _head_mat(att, in_dim):
    h, cdim = att.shape
    m = (att[:, :, None] * jnp.eye(h, dtype=att.dtype)[:, None, :]).reshape(in_dim, h)
    return jnp.pad(m, ((0, 0), (0, HW - h)))


def kernel(path_index, node_emb, W1, att_src1, att_dst1, W2, att_src2, att_dst2):
    src = path_index[0].astype(jnp.int32)
    dst = path_index[1].astype(jnp.int32)
    pad_ids = (N + (jnp.arange(EP - E, dtype=jnp.int32) % (NP - N)))
    src2d = jnp.concatenate([src, pad_ids]).reshape(NROWS, 128)
    dst2d = jnp.concatenate([dst, pad_ids]).reshape(NROWS, 128)

    xp = jnp.pad(node_emb, ((0, NP - N), (0, 0)))
    a1s = _head_mat(att_src1, 128)
    a1d = _head_mat(att_dst1, 128)
    a2s = _head_mat(att_src2, 512)
    a2d = _head_mat(att_dst2, 512)

    h1_0, h1_1, as1, ad1 = _dense1(xp, W1, a1s, a1d)
    o1_0, o1_1, _ex1 = _edge_l1(src2d, dst2d, as1, ad1, h1_0, h1_1)
    h2_0, h2_1, h2_2, h2_3, as2, ad2 = _dense2(
        o1_0, o1_1, W2[:64], W2[64:], a2s, a2d)
    o2_0, o2_1, o2_2, o2_3, _ex2 = _edge_l2(
        src2d, dst2d, as2, ad2, h2_0, h2_1, h2_2, h2_3)
    return jnp.concatenate([o2_0, o2_1, o2_2, o2_3], axis=1)[:N]
